# Initial kernel scaffold; baseline (speedup 1.0000x reference)
import jax
import jax.numpy as jnp
from jax.experimental import pallas as pl


def kernel(tensor, bank_weights, bank_selections, W, bias):
    raise NotImplementedError("write your pallas kernel here")


# speedup vs baseline: 4.7179x; 4.7179x over previous
import jax
import jax.numpy as jnp
from jax.experimental import pallas as pl
from jax.experimental.pallas import tpu as pltpu

B = 4
S = 2048
IN_F = 1024
OUT_F = 1024
NUM_BANKS = 16


def _body(sel_ref, bw_ref, x_ref, w0_ref, w1_ref, bias_ref, out_ref):
    b = pl.program_id(0)
    bw0 = bw_ref[b, 0]
    bw1 = bw_ref[b, 1]
    w_eff = bw0 * w0_ref[0] + bw1 * w1_ref[0]
    acc = jnp.dot(x_ref[0], w_eff, preferred_element_type=jnp.float32)
    s0 = sel_ref[b, 0]
    s1 = sel_ref[b, 1]
    b_eff = bw0 * bias_ref[s0, :] + bw1 * bias_ref[s1, :]
    out_ref[0] = acc + b_eff[None, :]


def kernel(tensor, bank_weights, bank_selections, W, bias):
    grid_spec = pltpu.PrefetchScalarGridSpec(
        num_scalar_prefetch=2,
        grid=(B,),
        in_specs=[
            pl.BlockSpec((1, S, IN_F), lambda b, sel, bw: (b, 0, 0)),
            pl.BlockSpec((1, IN_F, OUT_F), lambda b, sel, bw: (sel[b, 0], 0, 0)),
            pl.BlockSpec((1, IN_F, OUT_F), lambda b, sel, bw: (sel[b, 1], 0, 0)),
            pl.BlockSpec((NUM_BANKS, OUT_F), lambda b, sel, bw: (0, 0)),
        ],
        out_specs=pl.BlockSpec((1, S, OUT_F), lambda b, sel, bw: (b, 0, 0)),
    )
    return pl.pallas_call(
        _body,
        grid_spec=grid_spec,
        out_shape=jax.ShapeDtypeStruct((B, S, OUT_F), jnp.float32),
    )(bank_selections, bank_weights, tensor, W, W, bias)


# speedup vs baseline: 4.7384x; 1.0043x over previous
import jax
import jax.numpy as jnp
from jax.experimental import pallas as pl
from jax.experimental.pallas import tpu as pltpu

B = 4
S = 2048
IN_F = 1024
OUT_F = 1024
NUM_BANKS = 16


def _body(sel_ref, bw_ref, x_ref, w0_ref, w1_ref, bias_ref, out_ref):
    b = pl.program_id(0)
    bw0 = bw_ref[b, 0]
    bw1 = bw_ref[b, 1]
    w_eff = (bw0 * w0_ref[0] + bw1 * w1_ref[0]).astype(jnp.bfloat16)
    acc = jnp.dot(x_ref[0].astype(jnp.bfloat16), w_eff,
                  preferred_element_type=jnp.float32)
    s0 = sel_ref[b, 0]
    s1 = sel_ref[b, 1]
    b_eff = bw0 * bias_ref[s0, :] + bw1 * bias_ref[s1, :]
    out_ref[0] = acc + b_eff[None, :]


def kernel(tensor, bank_weights, bank_selections, W, bias):
    grid_spec = pltpu.PrefetchScalarGridSpec(
        num_scalar_prefetch=2,
        grid=(B,),
        in_specs=[
            pl.BlockSpec((1, S, IN_F), lambda b, sel, bw: (b, 0, 0)),
            pl.BlockSpec((1, IN_F, OUT_F), lambda b, sel, bw: (sel[b, 0], 0, 0)),
            pl.BlockSpec((1, IN_F, OUT_F), lambda b, sel, bw: (sel[b, 1], 0, 0)),
            pl.BlockSpec((NUM_BANKS, OUT_F), lambda b, sel, bw: (0, 0)),
        ],
        out_specs=pl.BlockSpec((1, S, OUT_F), lambda b, sel, bw: (b, 0, 0)),
    )
    return pl.pallas_call(
        _body,
        grid_spec=grid_spec,
        out_shape=jax.ShapeDtypeStruct((B, S, OUT_F), jnp.float32),
    )(bank_selections, bank_weights, tensor, W, W, bias)
